# Initial kernel scaffold; baseline (speedup 1.0000x reference)
import jax
import jax.numpy as jnp
from jax.experimental import pallas as pl


def kernel(t, xh, node_mask, edge_mask, params):
    raise NotImplementedError("write your pallas kernel here")


# speedup vs baseline: 22.6763x; 22.6763x over previous
import jax
import jax.numpy as jnp
from jax.experimental import pallas as pl

_HID = 64
_NLAYERS = 4
_NORM = 100.0


def _silu(x):
    return x * jax.nn.sigmoid(x)


def _gnn_kernel(nn, gpb, xh_ref, t_ref, nm_ref, wx_ref, wt_ref, bemb_ref,
                e1s_ref, e1d_ref, b1e_ref, e2_ref, b2e_ref,
                n1h_ref, n1a_ref, b1n_ref, n2_ref, b2n_ref,
                wout_ref, bout_ref, out_ref):
    H = _HID
    rows = gpb * nn
    nm = nm_ref[:, :]
    x = xh_ref[:, :] * nm
    hdd = (jnp.dot(x, wx_ref[:, :], preferred_element_type=jnp.float32)
           + t_ref[:, :] * wt_ref[:, :]
           + bemb_ref[:, :])
    for i in range(_NLAYERS):
        a = jnp.dot(hdd, e1s_ref[i], preferred_element_type=jnp.float32)
        b = (jnp.dot(hdd, e1d_ref[i], preferred_element_type=jnp.float32)
             + b1e_ref[i])
        ea = jnp.broadcast_to(a.reshape(gpb, nn, 1, H), (gpb, nn, nn, H))
        eb = jnp.broadcast_to(b.reshape(gpb, 1, nn, H), (gpb, nn, nn, H))
        h1 = _silu(ea + eb).reshape(gpb * nn * nn, H)
        m = _silu(jnp.dot(h1, e2_ref[i], preferred_element_type=jnp.float32)
                  + b2e_ref[i])
        agg = m.reshape(gpb, nn, nn, H).sum(axis=2) * (1.0 / _NORM)
        agg = agg.reshape(rows, H)
        c = (jnp.dot(hdd, n1h_ref[i], preferred_element_type=jnp.float32)
             + jnp.dot(agg, n1a_ref[i], preferred_element_type=jnp.float32)
             + b1n_ref[i])
        hdd = (hdd
               + jnp.dot(_silu(c), n2_ref[i], preferred_element_type=jnp.float32)
               + b2n_ref[i]) * nm
    out_ref[:, :] = (jnp.dot(hdd, wout_ref[:, :],
                             preferred_element_type=jnp.float32)
                     + bout_ref[:, :]) * nm


def kernel(t, xh, node_mask, edge_mask, params):
    bs, nn, dims = xh.shape
    N = bs * nn
    H = _HID
    gpb = 1
    grid = bs // gpb
    rows = gpb * nn

    xh9 = xh.reshape(N, dims)
    t_rep = jnp.repeat(t, nn).reshape(N, 1)
    nm = node_mask.reshape(N, 1)

    w_emb, b_emb = params['emb']
    wx = w_emb[:dims]
    wt = w_emb[dims:].reshape(1, H)
    bemb = b_emb.reshape(1, H)

    e1s = jnp.stack([params['gcl%d_e1' % i][0][:H] for i in range(_NLAYERS)])
    e1d = jnp.stack([params['gcl%d_e1' % i][0][H:] for i in range(_NLAYERS)])
    b1e = jnp.stack([params['gcl%d_e1' % i][1].reshape(1, H)
                     for i in range(_NLAYERS)])
    e2 = jnp.stack([params['gcl%d_e2' % i][0] for i in range(_NLAYERS)])
    b2e = jnp.stack([params['gcl%d_e2' % i][1].reshape(1, H)
                     for i in range(_NLAYERS)])
    n1h = jnp.stack([params['gcl%d_n1' % i][0][:H] for i in range(_NLAYERS)])
    n1a = jnp.stack([params['gcl%d_n1' % i][0][H:] for i in range(_NLAYERS)])
    b1n = jnp.stack([params['gcl%d_n1' % i][1].reshape(1, H)
                     for i in range(_NLAYERS)])
    n2 = jnp.stack([params['gcl%d_n2' % i][0] for i in range(_NLAYERS)])
    b2n = jnp.stack([params['gcl%d_n2' % i][1].reshape(1, H)
                     for i in range(_NLAYERS)])
    wout, bout = params['out']
    bout = bout.reshape(1, dims)

    row_spec = lambda w: pl.BlockSpec((rows, w), lambda i: (i, 0))
    full2 = lambda arr: pl.BlockSpec(arr.shape, lambda i: (0, 0))
    full3 = lambda arr: pl.BlockSpec(arr.shape, lambda i: (0, 0, 0))

    import functools
    out = pl.pallas_call(
        functools.partial(_gnn_kernel, nn, gpb),
        grid=(grid,),
        in_specs=[
            row_spec(dims), row_spec(1), row_spec(1),
            full2(wx), full2(wt), full2(bemb),
            full3(e1s), full3(e1d), full3(b1e), full3(e2), full3(b2e),
            full3(n1h), full3(n1a), full3(b1n), full3(n2), full3(b2n),
            full2(wout), full2(bout),
        ],
        out_specs=pl.BlockSpec((rows, dims), lambda i: (i, 0)),
        out_shape=jax.ShapeDtypeStruct((N, dims), jnp.float32),
    )(xh9, t_rep, nm, wx, wt, bemb,
      e1s, e1d, b1e, e2, b2e, n1h, n1a, b1n, n2, b2n, wout, bout)
    return out.reshape(bs, nn, dims)


# speedup vs baseline: 35.9122x; 1.5837x over previous
import jax
import jax.numpy as jnp
from jax.experimental import pallas as pl

_HID = 64
_NLAYERS = 4
_NORM = 100.0


def _silu(x):
    hx = 0.5 * x
    return hx * jnp.tanh(hx) + hx


def _gnn_kernel(nn, gpb, xh_ref, t_ref, nm_ref, wx_ref, wt_ref, bemb_ref,
                e1s_ref, e1d_ref, b1e_ref, e2_ref, b2e_ref,
                n1h_ref, n1a_ref, b1n_ref, n2_ref, b2n_ref,
                wout_ref, bout_ref, out_ref):
    H = _HID
    rows = gpb * nn
    nm = nm_ref[:, :]
    x = xh_ref[:, :] * nm
    hdd = (jnp.dot(x, wx_ref[:, :], preferred_element_type=jnp.float32)
           + t_ref[:, :] * wt_ref[:, :]
           + bemb_ref[:, :])
    for i in range(_NLAYERS):
        a = jnp.dot(hdd, e1s_ref[i], preferred_element_type=jnp.float32)
        b = (jnp.dot(hdd, e1d_ref[i], preferred_element_type=jnp.float32)
             + b1e_ref[i])
        ea = jnp.broadcast_to(a.reshape(gpb, nn, 1, H), (gpb, nn, nn, H))
        eb = jnp.broadcast_to(b.reshape(gpb, 1, nn, H), (gpb, nn, nn, H))
        h1 = _silu(ea + eb).reshape(gpb * nn * nn, H)
        m = _silu(jnp.dot(h1, e2_ref[i], preferred_element_type=jnp.float32)
                  + b2e_ref[i])
        agg = m.reshape(gpb, nn, nn, H).sum(axis=2) * (1.0 / _NORM)
        agg = agg.reshape(rows, H)
        c = (jnp.dot(hdd, n1h_ref[i], preferred_element_type=jnp.float32)
             + jnp.dot(agg, n1a_ref[i], preferred_element_type=jnp.float32)
             + b1n_ref[i])
        hdd = (hdd
               + jnp.dot(_silu(c), n2_ref[i], preferred_element_type=jnp.float32)
               + b2n_ref[i]) * nm
    out_ref[:, :] = (jnp.dot(hdd, wout_ref[:, :],
                             preferred_element_type=jnp.float32)
                     + bout_ref[:, :]) * nm


def kernel(t, xh, node_mask, edge_mask, params):
    bs, nn, dims = xh.shape
    N = bs * nn
    H = _HID
    gpb = 8
    grid = bs // gpb
    rows = gpb * nn

    xh9 = xh.reshape(N, dims)
    t_rep = jnp.repeat(t, nn).reshape(N, 1)
    nm = node_mask.reshape(N, 1)

    w_emb, b_emb = params['emb']
    wx = w_emb[:dims]
    wt = w_emb[dims:].reshape(1, H)
    bemb = b_emb.reshape(1, H)

    e1s = jnp.stack([params['gcl%d_e1' % i][0][:H] for i in range(_NLAYERS)])
    e1d = jnp.stack([params['gcl%d_e1' % i][0][H:] for i in range(_NLAYERS)])
    b1e = jnp.stack([params['gcl%d_e1' % i][1].reshape(1, H)
                     for i in range(_NLAYERS)])
    e2 = jnp.stack([params['gcl%d_e2' % i][0] for i in range(_NLAYERS)])
    b2e = jnp.stack([params['gcl%d_e2' % i][1].reshape(1, H)
                     for i in range(_NLAYERS)])
    n1h = jnp.stack([params['gcl%d_n1' % i][0][:H] for i in range(_NLAYERS)])
    n1a = jnp.stack([params['gcl%d_n1' % i][0][H:] for i in range(_NLAYERS)])
    b1n = jnp.stack([params['gcl%d_n1' % i][1].reshape(1, H)
                     for i in range(_NLAYERS)])
    n2 = jnp.stack([params['gcl%d_n2' % i][0] for i in range(_NLAYERS)])
    b2n = jnp.stack([params['gcl%d_n2' % i][1].reshape(1, H)
                     for i in range(_NLAYERS)])
    wout, bout = params['out']
    bout = bout.reshape(1, dims)

    row_spec = lambda w: pl.BlockSpec((rows, w), lambda i: (i, 0))
    full2 = lambda arr: pl.BlockSpec(arr.shape, lambda i: (0, 0))
    full3 = lambda arr: pl.BlockSpec(arr.shape, lambda i: (0, 0, 0))

    import functools
    out = pl.pallas_call(
        functools.partial(_gnn_kernel, nn, gpb),
        grid=(grid,),
        in_specs=[
            row_spec(dims), row_spec(1), row_spec(1),
            full2(wx), full2(wt), full2(bemb),
            full3(e1s), full3(e1d), full3(b1e), full3(e2), full3(b2e),
            full3(n1h), full3(n1a), full3(b1n), full3(n2), full3(b2n),
            full2(wout), full2(bout),
        ],
        out_specs=pl.BlockSpec((rows, dims), lambda i: (i, 0)),
        out_shape=jax.ShapeDtypeStruct((N, dims), jnp.float32),
    )(xh9, t_rep, nm, wx, wt, bemb,
      e1s, e1d, b1e, e2, b2e, n1h, n1a, b1n, n2, b2n, wout, bout)
    return out.reshape(bs, nn, dims)


# speedup vs baseline: 55.3186x; 1.5404x over previous
import jax
import jax.numpy as jnp
from jax.experimental import pallas as pl

_HID = 64
_NLAYERS = 4
_NORM = 100.0


def _silu(x):
    hx = 0.5 * x
    return hx * jnp.tanh(hx) + hx


def _gnn_kernel(nn, gpb, xh_ref, t_ref, nm_ref, wx_ref, wt_ref, bemb_ref,
                e1s_ref, e1d_ref, b1e_ref, e2_ref, b2e_ref,
                n1h_ref, n1a_ref, b1n_ref, n2_ref, b2n_ref,
                wout_ref, bout_ref, out_ref):
    H = _HID
    nh = nn // 2
    rows = gpb * nn
    nm = nm_ref[:, :]
    x = xh_ref[:, :] * nm
    hdd = (jnp.dot(x, wx_ref[:, :], preferred_element_type=jnp.float32)
           + t_ref[:, :] * wt_ref[:, :]
           + bemb_ref[:, :])
    for i in range(_NLAYERS):
        a = jnp.dot(hdd, e1s_ref[i], preferred_element_type=jnp.float32)
        b = (jnp.dot(hdd, e1d_ref[i], preferred_element_type=jnp.float32)
             + b1e_ref[i])
        a2 = jnp.concatenate([a, a], axis=1)
        b3 = b.reshape(gpb, nn, H)
        b2 = jnp.concatenate([b3[:, :nh, :], b3[:, nh:, :]], axis=2)
        ea = jnp.broadcast_to(a2.reshape(gpb, nn, 1, 2 * H),
                              (gpb, nn, nh, 2 * H))
        eb = jnp.broadcast_to(b2.reshape(gpb, 1, nh, 2 * H),
                              (gpb, nn, nh, 2 * H))
        h1 = _silu(ea + eb).reshape(gpb * nn * nh, 2 * H)
        m = _silu(jnp.dot(h1, e2_ref[i], preferred_element_type=jnp.float32)
                  + b2e_ref[i])
        s = m.reshape(gpb, nn, nh, 2 * H).sum(axis=2).reshape(rows, 2 * H)
        agg = (s[:, :H] + s[:, H:]) * (1.0 / _NORM)
        c = (jnp.dot(hdd, n1h_ref[i], preferred_element_type=jnp.float32)
             + jnp.dot(agg, n1a_ref[i], preferred_element_type=jnp.float32)
             + b1n_ref[i])
        hdd = (hdd
               + jnp.dot(_silu(c), n2_ref[i], preferred_element_type=jnp.float32)
               + b2n_ref[i]) * nm
    out_ref[:, :] = (jnp.dot(hdd, wout_ref[:, :],
                             preferred_element_type=jnp.float32)
                     + bout_ref[:, :]) * nm


def kernel(t, xh, node_mask, edge_mask, params):
    bs, nn, dims = xh.shape
    N = bs * nn
    H = _HID
    gpb = 8
    grid = bs // gpb
    rows = gpb * nn

    xh9 = xh.reshape(N, dims)
    t_rep = jnp.repeat(t, nn).reshape(N, 1)
    nm = node_mask.reshape(N, 1)

    w_emb, b_emb = params['emb']
    wx = w_emb[:dims]
    wt = w_emb[dims:].reshape(1, H)
    bemb = b_emb.reshape(1, H)

    e1s = jnp.stack([params['gcl%d_e1' % i][0][:H] for i in range(_NLAYERS)])
    e1d = jnp.stack([params['gcl%d_e1' % i][0][H:] for i in range(_NLAYERS)])
    b1e = jnp.stack([params['gcl%d_e1' % i][1].reshape(1, H)
                     for i in range(_NLAYERS)])
    zero = jnp.zeros((H, H), jnp.float32)
    e2 = jnp.stack([
        jnp.block([[params['gcl%d_e2' % i][0], zero],
                   [zero, params['gcl%d_e2' % i][0]]])
        for i in range(_NLAYERS)])
    b2e = jnp.stack([jnp.tile(params['gcl%d_e2' % i][1], 2).reshape(1, 2 * H)
                     for i in range(_NLAYERS)])
    n1h = jnp.stack([params['gcl%d_n1' % i][0][:H] for i in range(_NLAYERS)])
    n1a = jnp.stack([params['gcl%d_n1' % i][0][H:] for i in range(_NLAYERS)])
    b1n = jnp.stack([params['gcl%d_n1' % i][1].reshape(1, H)
                     for i in range(_NLAYERS)])
    n2 = jnp.stack([params['gcl%d_n2' % i][0] for i in range(_NLAYERS)])
    b2n = jnp.stack([params['gcl%d_n2' % i][1].reshape(1, H)
                     for i in range(_NLAYERS)])
    wout, bout = params['out']
    bout = bout.reshape(1, dims)

    row_spec = lambda w: pl.BlockSpec((rows, w), lambda i: (i, 0))
    full2 = lambda arr: pl.BlockSpec(arr.shape, lambda i: (0, 0))
    full3 = lambda arr: pl.BlockSpec(arr.shape, lambda i: (0, 0, 0))

    import functools
    out = pl.pallas_call(
        functools.partial(_gnn_kernel, nn, gpb),
        grid=(grid,),
        in_specs=[
            row_spec(dims), row_spec(1), row_spec(1),
            full2(wx), full2(wt), full2(bemb),
            full3(e1s), full3(e1d), full3(b1e), full3(e2), full3(b2e),
            full3(n1h), full3(n1a), full3(b1n), full3(n2), full3(b2n),
            full2(wout), full2(bout),
        ],
        out_specs=pl.BlockSpec((rows, dims), lambda i: (i, 0)),
        out_shape=jax.ShapeDtypeStruct((N, dims), jnp.float32),
    )(xh9, t_rep, nm, wx, wt, bemb,
      e1s, e1d, b1e, e2, b2e, n1h, n1a, b1n, n2, b2n, wout, bout)
    return out.reshape(bs, nn, dims)


# speedup vs baseline: 70.4487x; 1.2735x over previous
import jax
import jax.numpy as jnp
from jax.experimental import pallas as pl

_HID = 64
_NLAYERS = 4
_NORM = 100.0


def _silu2h(hx):
    return hx * jnp.tanh(hx) + hx


def _gnn_kernel(nn, gpb, xh_ref, t_ref, nm_ref, wx_ref, wt_ref, bemb_ref,
                e1s_ref, e1d_ref, b1e_ref, e2_ref, b2e_ref,
                n1h_ref, n1a_ref, b1n_ref, n2_ref, b2n_ref,
                wout_ref, bout_ref, out_ref):
    H = _HID
    nh = nn // 2
    rows = gpb * nn
    nm = nm_ref[:, :]
    x = xh_ref[:, :] * nm
    hdd = (jnp.dot(x, wx_ref[:, :], preferred_element_type=jnp.float32)
           + t_ref[:, :] * wt_ref[:, :]
           + bemb_ref[:, :])
    for i in range(_NLAYERS):
        a = jnp.dot(hdd, e1s_ref[i], preferred_element_type=jnp.float32)
        b = (jnp.dot(hdd, e1d_ref[i], preferred_element_type=jnp.float32)
             + b1e_ref[i])
        a2 = jnp.concatenate([a, a], axis=1)
        b3 = b.reshape(gpb, nn, H)
        b2 = jnp.concatenate([b3[:, :nh, :], b3[:, nh:, :]], axis=2)
        ea = jnp.broadcast_to(a2.reshape(gpb, 1, nn, 2 * H),
                              (gpb, nh, nn, 2 * H))
        eb = jnp.broadcast_to(b2.reshape(gpb, nh, 1, 2 * H),
                              (gpb, nh, nn, 2 * H))
        h1 = _silu2h(ea + eb).reshape(gpb * nn * nh, 2 * H)
        m = _silu2h(jnp.dot(h1, e2_ref[i], preferred_element_type=jnp.float32)
                    + b2e_ref[i])
        s = m.reshape(gpb, nh, nn, 2 * H).sum(axis=1).reshape(rows, 2 * H)
        agg = (s[:, :H] + s[:, H:]) * (1.0 / _NORM)
        c = (jnp.dot(hdd, n1h_ref[i], preferred_element_type=jnp.float32)
             + jnp.dot(agg, n1a_ref[i], preferred_element_type=jnp.float32)
             + b1n_ref[i])
        hdd = (hdd
               + jnp.dot(_silu2h(c), n2_ref[i],
                         preferred_element_type=jnp.float32)
               + b2n_ref[i]) * nm
    out_ref[:, :] = (jnp.dot(hdd, wout_ref[:, :],
                             preferred_element_type=jnp.float32)
                     + bout_ref[:, :]) * nm


def kernel(t, xh, node_mask, edge_mask, params):
    bs, nn, dims = xh.shape
    N = bs * nn
    H = _HID
    gpb = 8
    grid = bs // gpb
    rows = gpb * nn

    xh9 = xh.reshape(N, dims)
    t_rep = jnp.repeat(t, nn).reshape(N, 1)
    nm = node_mask.reshape(N, 1)

    w_emb, b_emb = params['emb']
    wx = w_emb[:dims]
    wt = w_emb[dims:].reshape(1, H)
    bemb = b_emb.reshape(1, H)

    e1s = 0.5 * jnp.stack([params['gcl%d_e1' % i][0][:H]
                           for i in range(_NLAYERS)])
    e1d = 0.5 * jnp.stack([params['gcl%d_e1' % i][0][H:]
                           for i in range(_NLAYERS)])
    b1e = 0.5 * jnp.stack([params['gcl%d_e1' % i][1].reshape(1, H)
                           for i in range(_NLAYERS)])
    zero = jnp.zeros((H, H), jnp.float32)
    e2 = 0.5 * jnp.stack([
        jnp.block([[params['gcl%d_e2' % i][0], zero],
                   [zero, params['gcl%d_e2' % i][0]]])
        for i in range(_NLAYERS)])
    b2e = 0.5 * jnp.stack([jnp.tile(params['gcl%d_e2' % i][1], 2)
                           .reshape(1, 2 * H) for i in range(_NLAYERS)])
    n1h = 0.5 * jnp.stack([params['gcl%d_n1' % i][0][:H]
                           for i in range(_NLAYERS)])
    n1a = 0.5 * jnp.stack([params['gcl%d_n1' % i][0][H:]
                           for i in range(_NLAYERS)])
    b1n = 0.5 * jnp.stack([params['gcl%d_n1' % i][1].reshape(1, H)
                           for i in range(_NLAYERS)])
    n2 = jnp.stack([params['gcl%d_n2' % i][0] for i in range(_NLAYERS)])
    b2n = jnp.stack([params['gcl%d_n2' % i][1].reshape(1, H)
                     for i in range(_NLAYERS)])
    wout, bout = params['out']
    bout = bout.reshape(1, dims)

    row_spec = lambda w: pl.BlockSpec((rows, w), lambda i: (i, 0))
    full2 = lambda arr: pl.BlockSpec(arr.shape, lambda i: (0, 0))
    full3 = lambda arr: pl.BlockSpec(arr.shape, lambda i: (0, 0, 0))

    import functools
    out = pl.pallas_call(
        functools.partial(_gnn_kernel, nn, gpb),
        grid=(grid,),
        in_specs=[
            row_spec(dims), row_spec(1), row_spec(1),
            full2(wx), full2(wt), full2(bemb),
            full3(e1s), full3(e1d), full3(b1e), full3(e2), full3(b2e),
            full3(n1h), full3(n1a), full3(b1n), full3(n2), full3(b2n),
            full2(wout), full2(bout),
        ],
        out_specs=pl.BlockSpec((rows, dims), lambda i: (i, 0)),
        out_shape=jax.ShapeDtypeStruct((N, dims), jnp.float32),
    )(xh9, t_rep, nm, wx, wt, bemb,
      e1s, e1d, b1e, e2, b2e, n1h, n1a, b1n, n2, b2n, wout, bout)
    return out.reshape(bs, nn, dims)
